# baseline (speedup 1.0000x reference)
import functools

import jax
import jax.numpy as jnp
from jax import lax
from jax.experimental import pallas as pl
from jax.experimental.pallas import tpu as pltpu
from jax.experimental.pallas import tpu_sc as plsc

B = 16384
L = 200
V = 26
VP = 32
CP = 128
WPS = L // 4
NC, NS, LANES = 2, 16, 16
NW = NC * NS
SPW = B // NW
CHUNK = 128
NCHUNK = SPW // CHUNK
GROUPS = CHUNK // LANES


def _sc_hist_kernel(xw_hbm, out_hbm, x_buf0, x_buf1, counts_buf, sem0, sem1):
  wid = lax.axis_index("s") * NC + lax.axis_index("c")
  wb = wid * SPW

  iota = lax.broadcasted_iota(jnp.int32, (LANES,), 0)
  ones = jnp.full((LANES,), 1.0, dtype=jnp.float32)
  zeros = jnp.zeros((LANES,), dtype=jnp.float32)
  sems = (sem0, sem1)
  bufs = (x_buf0, x_buf1)

  def start(c):
    return pltpu.async_copy(
        xw_hbm.at[pl.ds(wb + c * CHUNK, CHUNK), :],
        bufs[c % 2], sems[c % 2])

  cp = start(0)

  @plsc.parallel_loop(0, SPW, unroll=4)
  def _(i):
    counts_buf[i, pl.ds(0, LANES)] = zeros
    counts_buf[i, pl.ds(LANES, LANES)] = zeros

  for c in range(NCHUNK):
    nxt = start(c + 1) if c + 1 < NCHUNK else None
    cp.wait()
    buf = bufs[c % 2]

    def group_body(g, _):
      srow = g * LANES + iota
      rows = c * CHUNK + g * LANES + iota
      zero16 = jnp.zeros_like(iota)

      @plsc.parallel_loop(0, WPS, unroll=8)
      def _(lw):
        wv = plsc.load_gather(buf, [srow, zero16 + lw])
        for k in range(4):
          val = (wv >> (8 * k)) & 0xFF
          plsc.addupdate_scatter(counts_buf, [rows, val], ones)
      return 0
    lax.fori_loop(0, GROUPS, group_body, 0)
    cp = nxt

  pltpu.sync_copy(counts_buf, out_hbm.at[pl.ds(wb, SPW), :])


@jax.jit
def _sc_hist(x_words):
  mesh = plsc.VectorSubcoreMesh(core_axis_name="c", subcore_axis_name="s")
  fn = functools.partial(
      pl.kernel,
      mesh=mesh,
      compiler_params=pltpu.CompilerParams(needs_layout_passes=False),
      out_type=jax.ShapeDtypeStruct((B, CP), jnp.float32),
      scratch_types=[
          pltpu.VMEM((CHUNK, CP), jnp.int32),
          pltpu.VMEM((CHUNK, CP), jnp.int32),
          pltpu.VMEM((SPW, CP), jnp.float32),
          pltpu.SemaphoreType.DMA,
          pltpu.SemaphoreType.DMA,
      ],
  )(_sc_hist_kernel)
  return fn(x_words)


BLK = 4096


def _tc_mlp_kernel(counts_ref, emb_ref, w1_ref, b1_ref, w2_ref, b2_ref,
                   out_ref):
  cnt = counts_ref[...]
  lane = lax.broadcasted_iota(jnp.int32, cnt.shape, 1)
  cnt = jnp.where(lane < VP, cnt, 0.0)
  pooled = jnp.dot(cnt, emb_ref[...], precision=lax.Precision.HIGHEST,
                   preferred_element_type=jnp.float32)
  h = jnp.dot(pooled, w1_ref[...], preferred_element_type=jnp.float32)
  h = jnp.maximum(h + b1_ref[...], 0.0)
  out_ref[...] = (
      jnp.dot(h, w2_ref[...], preferred_element_type=jnp.float32)
      + b2_ref[...])


@jax.jit
def _tc_mlp(counts2d, emb_pad, W1, b1r, W2, b2r):
  d = emb_pad.shape[1]
  h = W1.shape[1]
  o = W2.shape[1]
  return pl.pallas_call(
      _tc_mlp_kernel,
      grid=(B // BLK,),
      in_specs=[
          pl.BlockSpec((BLK, CP), lambda i: (i, 0)),
          pl.BlockSpec((CP, d), lambda i: (0, 0)),
          pl.BlockSpec((d, h), lambda i: (0, 0)),
          pl.BlockSpec((1, h), lambda i: (0, 0)),
          pl.BlockSpec((h, o), lambda i: (0, 0)),
          pl.BlockSpec((1, o), lambda i: (0, 0)),
      ],
      out_specs=pl.BlockSpec((BLK, o), lambda i: (i, 0)),
      out_shape=jax.ShapeDtypeStruct((B, o), jnp.float32),
  )(counts2d, emb_pad, W1, b1r, W2, b2r)


def kernel(x, emb, W1, b1, W2, b2):
  q = WPS
  x = x.astype(jnp.int32)
  xw = (x[:, 0:q] | (x[:, q:2 * q] << 8) | (x[:, 2 * q:3 * q] << 16)
        | (x[:, 3 * q:4 * q] << 24))
  xw = jnp.pad(xw, ((0, 0), (0, CP - q)))
  counts = _sc_hist(xw)

  d = emb.shape[1]
  emb_pad = jnp.zeros((CP, d), jnp.float32).at[:V].set(emb)
  return _tc_mlp(counts, emb_pad, W1, b1.reshape(1, -1), W2,
                 b2.reshape(1, -1))


# speedup vs baseline: 1.0988x; 1.0988x over previous
import functools

import jax
import jax.numpy as jnp
from jax import lax
from jax.experimental import pallas as pl
from jax.experimental.pallas import tpu as pltpu
from jax.experimental.pallas import tpu_sc as plsc

B = 16384
L = 200
V = 26
VP = 32
CP = 128
WPS = L // 4
NC, NS, LANES = 2, 16, 16
NW = NC * NS
SPW = B // NW
CHUNK = 128
NCHUNK = SPW // CHUNK
GROUPS = CHUNK // LANES


def _sc_hist_kernel(xw_hbm, out_hbm, x_buf0, x_buf1, counts_buf, sem0, sem1):
  wid = lax.axis_index("s") * NC + lax.axis_index("c")
  wb = wid * SPW

  iota = lax.broadcasted_iota(jnp.int32, (LANES,), 0)
  ones = jnp.full((LANES,), 1.0, dtype=jnp.float32)
  zeros = jnp.zeros((LANES,), dtype=jnp.float32)
  sems = (sem0, sem1)
  bufs = (x_buf0, x_buf1)

  def start(c):
    return pltpu.async_copy(
        xw_hbm.at[pl.ds(wb + c * CHUNK, CHUNK), :],
        bufs[c % 2], sems[c % 2])

  cp = start(0)

  @plsc.parallel_loop(0, SPW, unroll=4)
  def _(i):
    counts_buf[i, pl.ds(0, LANES)] = zeros
    counts_buf[i, pl.ds(LANES, LANES)] = zeros

  for c in range(NCHUNK):
    nxt = start(c + 1) if c + 1 < NCHUNK else None
    cp.wait()
    buf = bufs[c % 2]

    def group_body(g, _):
      srow = g * LANES + iota
      rows = c * CHUNK + g * LANES + iota
      zero16 = jnp.zeros_like(iota)

      @plsc.parallel_loop(0, L - CP, unroll=4)
      def _(lw):
        wv = plsc.load_gather(buf, [srow, zero16 + lw])
        plsc.addupdate_scatter(counts_buf, [rows, wv & 0xFF], ones)
        plsc.addupdate_scatter(counts_buf, [rows, wv >> 8], ones)

      @plsc.parallel_loop(L - CP, CP, unroll=4)
      def _(lw):
        wv = plsc.load_gather(buf, [srow, zero16 + lw])
        plsc.addupdate_scatter(counts_buf, [rows, wv], ones)
      return 0
    lax.fori_loop(0, GROUPS, group_body, 0)
    cp = nxt

  pltpu.sync_copy(counts_buf, out_hbm.at[pl.ds(wb, SPW), :])


@jax.jit
def _sc_hist(x_words):
  mesh = plsc.VectorSubcoreMesh(core_axis_name="c", subcore_axis_name="s")
  fn = functools.partial(
      pl.kernel,
      mesh=mesh,
      compiler_params=pltpu.CompilerParams(needs_layout_passes=False),
      out_type=jax.ShapeDtypeStruct((B, CP), jnp.float32),
      scratch_types=[
          pltpu.VMEM((CHUNK, CP), jnp.int32),
          pltpu.VMEM((CHUNK, CP), jnp.int32),
          pltpu.VMEM((SPW, CP), jnp.float32),
          pltpu.SemaphoreType.DMA,
          pltpu.SemaphoreType.DMA,
      ],
  )(_sc_hist_kernel)
  return fn(x_words)


BLK = 4096


def _tc_mlp_kernel(counts_ref, emb_ref, w1_ref, b1_ref, w2_ref, b2_ref,
                   out_ref):
  cnt = counts_ref[...]
  lane = lax.broadcasted_iota(jnp.int32, cnt.shape, 1)
  cnt = jnp.where(lane < VP, cnt, 0.0)
  pooled = jnp.dot(cnt, emb_ref[...], precision=lax.Precision.HIGHEST,
                   preferred_element_type=jnp.float32)
  h = jnp.dot(pooled, w1_ref[...], preferred_element_type=jnp.float32)
  h = jnp.maximum(h + b1_ref[...], 0.0)
  out_ref[...] = (
      jnp.dot(h, w2_ref[...], preferred_element_type=jnp.float32)
      + b2_ref[...])


@jax.jit
def _tc_mlp(counts2d, emb_pad, W1, b1r, W2, b2r):
  d = emb_pad.shape[1]
  h = W1.shape[1]
  o = W2.shape[1]
  return pl.pallas_call(
      _tc_mlp_kernel,
      grid=(B // BLK,),
      in_specs=[
          pl.BlockSpec((BLK, CP), lambda i: (i, 0)),
          pl.BlockSpec((CP, d), lambda i: (0, 0)),
          pl.BlockSpec((d, h), lambda i: (0, 0)),
          pl.BlockSpec((1, h), lambda i: (0, 0)),
          pl.BlockSpec((h, o), lambda i: (0, 0)),
          pl.BlockSpec((1, o), lambda i: (0, 0)),
      ],
      out_specs=pl.BlockSpec((BLK, o), lambda i: (i, 0)),
      out_shape=jax.ShapeDtypeStruct((B, o), jnp.float32),
  )(counts2d, emb_pad, W1, b1r, W2, b2r)


def kernel(x, emb, W1, b1, W2, b2):
  x = x.astype(jnp.int32)
  xw = x[:, :CP] | (jnp.pad(x[:, CP:], ((0, 0), (0, 2 * CP - L))) << 8)
  counts = _sc_hist(xw)

  d = emb.shape[1]
  emb_pad = jnp.zeros((CP, d), jnp.float32).at[:V].set(emb)
  return _tc_mlp(counts, emb_pad, W1, b1.reshape(1, -1), W2,
                 b2.reshape(1, -1))


# speedup vs baseline: 1.1393x; 1.0369x over previous
import functools

import jax
import jax.numpy as jnp
from jax import lax
from jax.experimental import pallas as pl
from jax.experimental.pallas import tpu as pltpu
from jax.experimental.pallas import tpu_sc as plsc

B = 16384
L = 200
V = 26
VP = 32
CP = 128
WPS = L // 4
NC, NS, LANES = 2, 16, 16
NW = NC * NS
SPW = B // NW
CHUNK = 128
NCHUNK = SPW // CHUNK
GROUPS = CHUNK // LANES
STRIDE = CP + 1


def _sc_hist_kernel(xw_hbm, out_hbm, x_buf0, x_buf1, counts_buf, sem0, sem1):
  wid = lax.axis_index("s") * NC + lax.axis_index("c")
  wb = wid * SPW

  iota = lax.broadcasted_iota(jnp.int32, (LANES,), 0)
  ones = jnp.full((LANES,), 1.0, dtype=jnp.float32)
  zeros = jnp.zeros((LANES,), dtype=jnp.float32)
  sems = (sem0, sem1)
  bufs = (x_buf0, x_buf1)

  def start(c):
    return pltpu.async_copy(
        xw_hbm.at[pl.ds(wb + c * CHUNK, CHUNK), :],
        bufs[c % 2].at[:, pl.ds(0, CP)], sems[c % 2])

  cp = start(0)

  @plsc.parallel_loop(0, SPW, unroll=4)
  def _(i):
    counts_buf[i, pl.ds(0, LANES)] = zeros
    counts_buf[i, pl.ds(LANES, LANES)] = zeros

  for c in range(NCHUNK):
    nxt = start(c + 1) if c + 1 < NCHUNK else None
    cp.wait()
    buf = bufs[c % 2]

    def group_body(g, _):
      srow = g * LANES + iota
      rows = c * CHUNK + g * LANES + iota
      zero16 = jnp.zeros_like(iota)

      @plsc.parallel_loop(0, L - CP, unroll=4)
      def _(lw):
        wv = plsc.load_gather(buf, [srow, zero16 + lw])
        plsc.addupdate_scatter(counts_buf, [rows, wv & 0xFF], ones)
        plsc.addupdate_scatter(counts_buf, [rows, wv >> 8], ones)

      @plsc.parallel_loop(L - CP, CP, unroll=4)
      def _(lw):
        wv = plsc.load_gather(buf, [srow, zero16 + lw])
        plsc.addupdate_scatter(counts_buf, [rows, wv], ones)
      return 0
    lax.fori_loop(0, GROUPS, group_body, 0)
    cp = nxt

  pltpu.sync_copy(counts_buf, out_hbm.at[pl.ds(wb, SPW), :])


@jax.jit
def _sc_hist(x_words):
  mesh = plsc.VectorSubcoreMesh(core_axis_name="c", subcore_axis_name="s")
  fn = functools.partial(
      pl.kernel,
      mesh=mesh,
      compiler_params=pltpu.CompilerParams(needs_layout_passes=False),
      out_type=jax.ShapeDtypeStruct((B, CP), jnp.float32),
      scratch_types=[
          pltpu.VMEM((CHUNK, STRIDE), jnp.int32),
          pltpu.VMEM((CHUNK, STRIDE), jnp.int32),
          pltpu.VMEM((SPW, CP), jnp.float32),
          pltpu.SemaphoreType.DMA,
          pltpu.SemaphoreType.DMA,
      ],
  )(_sc_hist_kernel)
  return fn(x_words)


PBLK = 4096


def _tc_pack_kernel(x_ref, out_ref):
  lo = x_ref[:, :CP]
  hi = jnp.concatenate(
      [x_ref[:, CP:], jnp.zeros((PBLK, 2 * CP - L), jnp.int32)], axis=1)
  out_ref[...] = lo | (hi << 8)


@jax.jit
def _tc_pack(x):
  return pl.pallas_call(
      _tc_pack_kernel,
      grid=(B // PBLK,),
      in_specs=[pl.BlockSpec((PBLK, L), lambda i: (i, 0))],
      out_specs=pl.BlockSpec((PBLK, CP), lambda i: (i, 0)),
      out_shape=jax.ShapeDtypeStruct((B, CP), jnp.int32),
  )(x)


BLK = 4096


def _tc_mlp_kernel(counts_ref, emb_ref, w1_ref, b1_ref, w2_ref, b2_ref,
                   out_ref):
  cnt = counts_ref[...]
  lane = lax.broadcasted_iota(jnp.int32, cnt.shape, 1)
  cnt = jnp.where(lane < VP, cnt, 0.0)
  pooled = jnp.dot(cnt, emb_ref[...], precision=lax.Precision.HIGHEST,
                   preferred_element_type=jnp.float32)
  h = jnp.dot(pooled, w1_ref[...], preferred_element_type=jnp.float32)
  h = jnp.maximum(h + b1_ref[...], 0.0)
  out_ref[...] = (
      jnp.dot(h, w2_ref[...], preferred_element_type=jnp.float32)
      + b2_ref[...])


@jax.jit
def _tc_mlp(counts2d, emb_pad, W1, b1r, W2, b2r):
  d = emb_pad.shape[1]
  h = W1.shape[1]
  o = W2.shape[1]
  return pl.pallas_call(
      _tc_mlp_kernel,
      grid=(B // BLK,),
      in_specs=[
          pl.BlockSpec((BLK, CP), lambda i: (i, 0)),
          pl.BlockSpec((CP, d), lambda i: (0, 0)),
          pl.BlockSpec((d, h), lambda i: (0, 0)),
          pl.BlockSpec((1, h), lambda i: (0, 0)),
          pl.BlockSpec((h, o), lambda i: (0, 0)),
          pl.BlockSpec((1, o), lambda i: (0, 0)),
      ],
      out_specs=pl.BlockSpec((BLK, o), lambda i: (i, 0)),
      out_shape=jax.ShapeDtypeStruct((B, o), jnp.float32),
  )(counts2d, emb_pad, W1, b1r, W2, b2r)


def kernel(x, emb, W1, b1, W2, b2):
  counts = _sc_hist(_tc_pack(x.astype(jnp.int32)))

  d = emb.shape[1]
  emb_pad = jnp.zeros((CP, d), jnp.float32).at[:V].set(emb)
  return _tc_mlp(counts, emb_pad, W1, b1.reshape(1, -1), W2,
                 b2.reshape(1, -1))


# speedup vs baseline: 1.2006x; 1.0538x over previous
import functools

import jax
import jax.numpy as jnp
from jax import lax
from jax.experimental import pallas as pl
from jax.experimental.pallas import tpu as pltpu
from jax.experimental.pallas import tpu_sc as plsc

B = 16384
L = 200
V = 26
VP = 32
CP = 128
WPS = L // 4
WROWS = B * WPS // CP
NC, NS, LANES = 2, 16, 16
NW = NC * NS
SPW = B // NW
GROUPS = SPW // LANES


def _sc_hist_kernel(xw_hbm, out_hbm, x_buf, counts_buf, sem0):
  wid = lax.axis_index("s") * NC + lax.axis_index("c")
  wb = wid * SPW

  iota = lax.broadcasted_iota(jnp.int32, (LANES,), 0)
  ones = jnp.full((LANES,), 1.0, dtype=jnp.float32)
  zeros = jnp.zeros((LANES,), dtype=jnp.float32)

  rows_per_w = SPW * WPS // CP
  cp = pltpu.async_copy(
      xw_hbm.at[pl.ds(wid * rows_per_w, rows_per_w), :], x_buf, sem0)

  @plsc.parallel_loop(0, SPW, unroll=4)
  def _(i):
    counts_buf[i, pl.ds(0, LANES)] = zeros
    counts_buf[i, pl.ds(LANES, LANES)] = zeros

  cp.wait()

  def group_body(g, _):
    jb = g >> 3
    lane0 = (g & 7) * LANES
    rows = g * LANES + iota

    @plsc.parallel_loop(0, WPS, unroll=4)
    def _(lw):
      wv = x_buf[jb * WPS + lw, pl.ds(lane0, LANES)]
      for k in range(4):
        val = (wv >> (8 * k)) & 0xFF
        plsc.addupdate_scatter(counts_buf, [rows, val], ones)
    return 0
  lax.fori_loop(0, GROUPS, group_body, 0)

  pltpu.sync_copy(counts_buf, out_hbm.at[pl.ds(wb, SPW), :])


@jax.jit
def _sc_hist(x_words):
  mesh = plsc.VectorSubcoreMesh(core_axis_name="c", subcore_axis_name="s")
  fn = functools.partial(
      pl.kernel,
      mesh=mesh,
      compiler_params=pltpu.CompilerParams(needs_layout_passes=False),
      out_type=jax.ShapeDtypeStruct((B, CP), jnp.float32),
      scratch_types=[
          pltpu.VMEM((SPW * WPS // CP, CP), jnp.int32),
          pltpu.VMEM((SPW, CP), jnp.float32),
          pltpu.SemaphoreType.DMA,
      ],
  )(_sc_hist_kernel)
  return fn(x_words)


PBLK = 1024


def _tc_pack_kernel(x_ref, out_ref):
  q = WPS
  x = x_ref[...]
  w = (x[:, 0:q] | (x[:, q:2 * q] << 8) | (x[:, 2 * q:3 * q] << 16)
       | (x[:, 3 * q:4 * q] << 24))
  wt = w.T
  for j in range(PBLK // CP):
    out_ref[pl.ds(q * j, q), :] = wt[:, CP * j:CP * (j + 1)]


@jax.jit
def _tc_pack(x):
  return pl.pallas_call(
      _tc_pack_kernel,
      grid=(B // PBLK,),
      in_specs=[pl.BlockSpec((PBLK, L), lambda i: (i, 0))],
      out_specs=pl.BlockSpec((PBLK * WPS // CP, CP), lambda i: (i, 0)),
      out_shape=jax.ShapeDtypeStruct((WROWS, CP), jnp.int32),
  )(x)


BLK = 4096


def _tc_mlp_kernel(counts_ref, emb_ref, w1_ref, b1_ref, w2_ref, b2_ref,
                   out_ref):
  cnt = counts_ref[...]
  lane = lax.broadcasted_iota(jnp.int32, cnt.shape, 1)
  cnt = jnp.where(lane < VP, cnt, 0.0)
  pooled = jnp.dot(cnt, emb_ref[...], precision=lax.Precision.HIGHEST,
                   preferred_element_type=jnp.float32)
  h = jnp.dot(pooled, w1_ref[...], preferred_element_type=jnp.float32)
  h = jnp.maximum(h + b1_ref[...], 0.0)
  out_ref[...] = (
      jnp.dot(h, w2_ref[...], preferred_element_type=jnp.float32)
      + b2_ref[...])


@jax.jit
def _tc_mlp(counts2d, emb_pad, W1, b1r, W2, b2r):
  d = emb_pad.shape[1]
  h = W1.shape[1]
  o = W2.shape[1]
  return pl.pallas_call(
      _tc_mlp_kernel,
      grid=(B // BLK,),
      in_specs=[
          pl.BlockSpec((BLK, CP), lambda i: (i, 0)),
          pl.BlockSpec((CP, d), lambda i: (0, 0)),
          pl.BlockSpec((d, h), lambda i: (0, 0)),
          pl.BlockSpec((1, h), lambda i: (0, 0)),
          pl.BlockSpec((h, o), lambda i: (0, 0)),
          pl.BlockSpec((1, o), lambda i: (0, 0)),
      ],
      out_specs=pl.BlockSpec((BLK, o), lambda i: (i, 0)),
      out_shape=jax.ShapeDtypeStruct((B, o), jnp.float32),
  )(counts2d, emb_pad, W1, b1r, W2, b2r)


def kernel(x, emb, W1, b1, W2, b2):
  counts = _sc_hist(_tc_pack(x))
  emb_pad = jnp.zeros((CP, emb.shape[1]), jnp.float32).at[:V].set(emb)
  return _tc_mlp(counts, emb_pad, W1, b1.reshape(1, -1), W2,
                 b2.reshape(1, -1))
